# baseline (speedup 1.0000x reference)
import functools

import jax
import jax.numpy as jnp
from jax import lax
from jax.experimental import pallas as pl
from jax.experimental.pallas import tpu as pltpu
from jax.experimental.pallas import tpu_sc as plsc

B = 16384
F = 26
K = 16
NC = 2
NS = 16
NW = NC * NS
ITEMS_W = B // NW
ROWS_W = ITEMS_W * F
GI = 16
CI = 64
ROWS_C = CI * F
NCH = ITEMS_W // CI
GPC = CI // GI

V = 2600000
TCH = 1024
NTCH = V // TCH
TAIL = V - NTCH * TCH
QMAX = (NTCH + 1 + NW - 1) // NW

_mesh = plsc.VectorSubcoreMesh(core_axis_name="c", subcore_axis_name="s")


@functools.partial(
    pl.kernel,
    out_type=jax.ShapeDtypeStruct((V * K,), jnp.float32),
    mesh=_mesh,
    compiler_params=pltpu.CompilerParams(
        needs_layout_passes=False, use_tc_tiling_on_sc=True),
    scratch_types=[
        pltpu.VMEM((8, TCH), jnp.float32),
        pltpu.VMEM((8, TCH), jnp.float32),
        pltpu.VMEM((8, TCH), jnp.float32),
        pltpu.VMEM((8, TCH), jnp.float32),
        pltpu.VMEM((K * TCH,), jnp.float32),
        pltpu.VMEM((K * TCH,), jnp.float32),
        pltpu.VMEM((K, TAIL), jnp.float32),
        pltpu.VMEM((K * TAIL,), jnp.float32),
        pltpu.SemaphoreType.DMA,
        pltpu.SemaphoreType.DMA,
        pltpu.SemaphoreType.DMA,
        pltpu.SemaphoreType.DMA,
    ],
)
def _transpose_sc(embt_hbm, out_hbm, inA0, inB0, inA1, inB1, ob0, ob1,
                  itail, otail, semi0, semi1, semo0, semo1):
    wid = lax.axis_index("s") * NC + lax.axis_index("c")
    lanes = lax.iota(jnp.int32, 16)
    LPC = TCH // 8
    kbase = lanes * TCH

    def bufs(buf):
        if buf == 0:
            return inA0, inB0, ob0, semi0, semo0
        return inA1, inB1, ob1, semi1, semo1

    def fire_in(c, buf):
        inA, inB, obuf, semi, semo = bufs(buf)
        pltpu.async_copy(embt_hbm.at[pl.ds(0, 8), pl.ds(c * TCH, TCH)],
                         inA, semi)
        pltpu.async_copy(embt_hbm.at[pl.ds(8, 8), pl.ds(c * TCH, TCH)],
                         inB, semi)

    def wait_in(c, buf):
        inA, inB, obuf, semi, semo = bufs(buf)
        pltpu.make_async_copy(embt_hbm.at[pl.ds(0, 8), pl.ds(c * TCH, TCH)],
                              inA, semi).wait()
        pltpu.make_async_copy(embt_hbm.at[pl.ds(8, 8), pl.ds(c * TCH, TCH)],
                              inB, semi).wait()

    def wait_out(c, buf):
        inA, inB, obuf, semi, semo = bufs(buf)
        pltpu.make_async_copy(obuf, out_hbm.at[pl.ds(c * K * TCH, K * TCH)],
                              semo).wait()

    def work(c, buf):
        inA, inB, obuf, semi, semo = bufs(buf)

        posbase = lanes * K

        @plsc.parallel_loop(0, TCH // K, unroll=1)
        def _(b):
            for k in range(K):
                ref = inA if k < 8 else inB
                v = ref[k % 8, pl.ds(b * K, K)]
                plsc.store_scatter(obuf, [posbase + (b * 256 + k)], v)

        pltpu.async_copy(obuf, out_hbm.at[pl.ds(c * K * TCH, K * TCH)], semo)

    fire_in(wid, 0)

    def q_step(q, buf):
        c = wid + q * NW
        cn = c + NW

        @pl.when(cn < NTCH)
        def _():
            fire_in(cn, 1 - buf)

        @pl.when(c < NTCH)
        def _():
            wait_in(c, buf)

            @pl.when(c >= 2 * NW)
            def _():
                wait_out(c - 2 * NW, buf)

            work(c, buf)

    def pair_body(p, _):
        q_step(2 * p, 0)
        q_step(2 * p + 1, 1)
        return 0

    lax.fori_loop(0, QMAX // 2, pair_body, 0)

    @pl.when(wid == NTCH % NW)
    def _():
        pltpu.sync_copy(embt_hbm.at[:, pl.ds(NTCH * TCH, TAIL)], itail)

        @plsc.parallel_loop(0, TAIL // 8, unroll=2)
        def _(m):
            for r in range(8):
                iv = jnp.full((16,), m * 8 + r, jnp.int32)
                otail[pl.ds(m * 128 + r * K, K)] = plsc.load_gather(
                    itail, [lanes, iv])

        pltpu.sync_copy(otail, out_hbm.at[pl.ds(NTCH * K * TCH, K * TAIL)])

    qlast = (NTCH - 1 - wid) // NW

    def drain_one(q):
        @pl.when(q >= 0)
        def _():
            c = wid + q * NW

            @pl.when(q % 2 == 0)
            def _():
                wait_out(c, 0)

            @pl.when(q % 2 == 1)
            def _():
                wait_out(c, 1)

    drain_one(qlast)
    drain_one(qlast - 1)


@functools.partial(
    pl.kernel,
    out_type=jax.ShapeDtypeStruct((B,), jnp.float32),
    mesh=_mesh,
    compiler_params=pltpu.CompilerParams(
        needs_layout_passes=False, use_tc_tiling_on_sc=False),
    scratch_types=[
        pltpu.VMEM((NCH, ROWS_C), jnp.int32),
        pltpu.VMEM((2, ROWS_C, K), jnp.float32),
        pltpu.VMEM((ROWS_C, 1), jnp.float32),
        pltpu.VMEM((ROWS_C, 1), jnp.float32),
        pltpu.VMEM((ITEMS_W,), jnp.float32),
        pltpu.VMEM((16,), jnp.float32),
        pltpu.SemaphoreType.DMA,
        pltpu.SemaphoreType.DMA,
    ],
)
def _fm_sc(x_hbm, emb_hbm, fc_hbm, wb_hbm, out_hbm,
           idx_v, rows_v, fcv0, fcv1, outbuf, wbv, sem0, sem1):
    wid = lax.axis_index("s") * NC + lax.axis_index("c")

    pltpu.sync_copy(x_hbm.at[wid], idx_v)
    pltpu.sync_copy(wb_hbm, wbv)
    wb16 = wbv[:]
    w = wb16[0]
    bias = wb16[1]

    lanes = lax.iota(jnp.int32, 16)

    _dn = lax.GatherDimensionNumbers(
        offset_dims=(), collapsed_slice_dims=(0,), start_index_map=(0,))

    def allsum16(v):
        for sh in (8, 4, 2, 1):
            perm = lax.gather(v, (lanes ^ sh)[:, None], dimension_numbers=_dn,
                              slice_sizes=(1,),
                              mode=lax.GatherScatterMode.PROMISE_IN_BOUNDS)
            v = v + perm
        return v

    def fire(c, buf):
        rbuf = rows_v.at[buf]
        fbuf = fcv0 if buf == 0 else fcv1
        sem = sem0 if buf == 0 else sem1
        pltpu.async_copy(emb_hbm.at[idx_v.at[c]], rbuf, sem)
        pltpu.async_copy(fc_hbm.at[idx_v.at[c]], fbuf, sem)

    def drain(c, buf):
        rbuf = rows_v.at[buf]
        fbuf = fcv0 if buf == 0 else fcv1
        sem = sem0 if buf == 0 else sem1
        pltpu.make_async_copy(emb_hbm.at[idx_v.at[c]], rbuf, sem).wait()
        pltpu.make_async_copy(fc_hbm.at[idx_v.at[c]], fbuf, sem).wait()

    def compute(c, buf):
        rbuf = rows_v.at[buf]
        fbuf = fcv0 if buf == 0 else fcv1

        for g in range(GPC):
            def item_body(b, pv, g=g):
                r0 = g * GI * F + b * F
                s = rbuf[r0, :]
                ss = s * s
                for f in range(1, F):
                    e = rbuf[r0 + f, :]
                    s = s + e
                    ss = ss + e * e
                pair = 0.5 * allsum16(s * s - ss)
                return jnp.where(lanes == b, pair, pv)

            pairvec = plsc.parallel_loop(
                0, GI, carry=jnp.zeros((16,), jnp.float32))(item_body)

            rowbase = g * GI * F + lanes * F
            zeros16 = jnp.zeros((16,), jnp.int32)
            fcs = jnp.zeros((16,), jnp.float32)
            for f in range(F):
                fcs = fcs + plsc.load_gather(fbuf, [rowbase + f, zeros16])

            z = pairvec + fcs * w + bias
            outbuf[pl.ds(c * CI + g * GI, GI)] = 1.0 / (1.0 + jnp.exp(-z))

    fire(0, 0)

    def pair_body(p, _):
        c0 = 2 * p
        fire(c0 + 1, 1)
        drain(c0, 0)
        compute(c0, 0)

        @pl.when(c0 + 2 < NCH)
        def _():
            fire(c0 + 2, 0)

        drain(c0 + 1, 1)
        compute(c0 + 1, 1)
        return 0

    lax.fori_loop(0, NCH // 2, pair_body, 0)

    pltpu.sync_copy(outbuf, out_hbm.at[pl.ds(wid * ITEMS_W, ITEMS_W)])


def kernel(x, emb_table, fc_table, lin_w, lin_b):
    x3 = x.reshape(NW, NCH, ROWS_C)
    emb_lin = _transpose_sc(emb_table.T)
    wb = jnp.zeros((16,), jnp.float32)
    wb = wb.at[0].set(lin_w[0, 0]).at[1].set(lin_b[0])
    out = _fm_sc(x3, emb_lin.reshape(V, K), fc_table, wb)
    return out.reshape(B, 1)


# speedup vs baseline: 22.6822x; 22.6822x over previous
import functools

import jax
import jax.numpy as jnp
from jax import lax
from jax.experimental import pallas as pl
from jax.experimental.pallas import tpu as pltpu
from jax.experimental.pallas import tpu_sc as plsc

B = 16384
F = 26
K = 16
NC = 2
NS = 16
NW = NC * NS
ITEMS_W = B // NW
ROWS_W = ITEMS_W * F
GI = 16
CI = 64
ROWS_C = CI * F
NCH = ITEMS_W // CI
GPC = CI // GI

V = 2600000
TCH = 1024
NTCH = V // TCH
TAIL = V - NTCH * TCH
QMAX = (NTCH + 1 + NW - 1) // NW

_mesh = plsc.VectorSubcoreMesh(core_axis_name="c", subcore_axis_name="s")


@functools.partial(
    pl.kernel,
    out_type=jax.ShapeDtypeStruct((V * K,), jnp.float32),
    mesh=_mesh,
    compiler_params=pltpu.CompilerParams(
        needs_layout_passes=False, use_tc_tiling_on_sc=True),
    scratch_types=[
        pltpu.VMEM((8, TCH), jnp.float32),
        pltpu.VMEM((8, TCH), jnp.float32),
        pltpu.VMEM((8, TCH), jnp.float32),
        pltpu.VMEM((8, TCH), jnp.float32),
        pltpu.VMEM((K * TCH,), jnp.float32),
        pltpu.VMEM((K * TCH,), jnp.float32),
        pltpu.VMEM((K, TAIL), jnp.float32),
        pltpu.VMEM((K * TAIL,), jnp.float32),
        pltpu.SemaphoreType.DMA,
        pltpu.SemaphoreType.DMA,
        pltpu.SemaphoreType.DMA,
        pltpu.SemaphoreType.DMA,
    ],
)
def _transpose_sc(embt_hbm, out_hbm, inA0, inB0, inA1, inB1, ob0, ob1,
                  itail, otail, semi0, semi1, semo0, semo1):
    wid = lax.axis_index("s") * NC + lax.axis_index("c")
    lanes = lax.iota(jnp.int32, 16)
    LPC = TCH // 8
    kbase = lanes * TCH

    def bufs(buf):
        if buf == 0:
            return inA0, inB0, ob0, semi0, semo0
        return inA1, inB1, ob1, semi1, semo1

    def fire_in(c, buf):
        inA, inB, obuf, semi, semo = bufs(buf)
        pltpu.async_copy(embt_hbm.at[pl.ds(0, 8), pl.ds(c * TCH, TCH)],
                         inA, semi)
        pltpu.async_copy(embt_hbm.at[pl.ds(8, 8), pl.ds(c * TCH, TCH)],
                         inB, semi)

    def wait_in(c, buf):
        inA, inB, obuf, semi, semo = bufs(buf)
        pltpu.make_async_copy(embt_hbm.at[pl.ds(0, 8), pl.ds(c * TCH, TCH)],
                              inA, semi).wait()
        pltpu.make_async_copy(embt_hbm.at[pl.ds(8, 8), pl.ds(c * TCH, TCH)],
                              inB, semi).wait()

    def wait_out(c, buf):
        inA, inB, obuf, semi, semo = bufs(buf)
        pltpu.make_async_copy(obuf, out_hbm.at[pl.ds(c * K * TCH, K * TCH)],
                              semo).wait()

    def work(c, buf):
        inA, inB, obuf, semi, semo = bufs(buf)

        posbase = lanes * K

        @plsc.parallel_loop(0, TCH // K, unroll=1)
        def _(b):
            for k in range(K):
                ref = inA if k < 8 else inB
                v = ref[k % 8, pl.ds(b * K, K)]
                plsc.store_scatter(obuf, [posbase + (b * 256 + k)], v)

        pltpu.async_copy(obuf, out_hbm.at[pl.ds(c * K * TCH, K * TCH)], semo)

    fire_in(wid, 0)

    def q_step(q, buf):
        c = wid + q * NW
        cn = c + NW

        @pl.when(cn < NTCH)
        def _():
            fire_in(cn, 1 - buf)

        @pl.when(c < NTCH)
        def _():
            wait_in(c, buf)

            @pl.when(c >= 2 * NW)
            def _():
                wait_out(c - 2 * NW, buf)

            work(c, buf)

    def pair_body(p, _):
        q_step(2 * p, 0)
        q_step(2 * p + 1, 1)
        return 0

    lax.fori_loop(0, QMAX // 2, pair_body, 0)

    @pl.when(wid == NTCH % NW)
    def _():
        pltpu.sync_copy(embt_hbm.at[:, pl.ds(NTCH * TCH, TAIL)], itail)

        @plsc.parallel_loop(0, TAIL // 8, unroll=2)
        def _(m):
            for r in range(8):
                iv = jnp.full((16,), m * 8 + r, jnp.int32)
                otail[pl.ds(m * 128 + r * K, K)] = plsc.load_gather(
                    itail, [lanes, iv])

        pltpu.sync_copy(otail, out_hbm.at[pl.ds(NTCH * K * TCH, K * TAIL)])

    qlast = (NTCH - 1 - wid) // NW

    def drain_one(q):
        @pl.when(q >= 0)
        def _():
            c = wid + q * NW

            @pl.when(q % 2 == 0)
            def _():
                wait_out(c, 0)

            @pl.when(q % 2 == 1)
            def _():
                wait_out(c, 1)

    drain_one(qlast)
    drain_one(qlast - 1)


@functools.partial(
    pl.kernel,
    out_type=jax.ShapeDtypeStruct((B,), jnp.float32),
    mesh=_mesh,
    compiler_params=pltpu.CompilerParams(
        needs_layout_passes=False, use_tc_tiling_on_sc=False),
    scratch_types=[
        pltpu.VMEM((NCH, ROWS_C), jnp.int32),
        pltpu.VMEM((2, ROWS_C, K), jnp.float32),
        pltpu.VMEM((ROWS_C,), jnp.float32),
        pltpu.VMEM((ROWS_C,), jnp.float32),
        pltpu.VMEM((ITEMS_W,), jnp.float32),
        pltpu.VMEM((16,), jnp.float32),
        pltpu.SemaphoreType.DMA,
        pltpu.SemaphoreType.DMA,
    ],
)
def _fm_sc(x_hbm, emb_hbm, fc_hbm, wb_hbm, out_hbm,
           idx_v, rows_v, fcv0, fcv1, outbuf, wbv, sem0, sem1):
    wid = lax.axis_index("s") * NC + lax.axis_index("c")

    pltpu.sync_copy(x_hbm.at[wid], idx_v)
    pltpu.sync_copy(wb_hbm, wbv)
    wb16 = wbv[:]
    w = wb16[0]
    bias = wb16[1]

    lanes = lax.iota(jnp.int32, 16)

    _dn = lax.GatherDimensionNumbers(
        offset_dims=(), collapsed_slice_dims=(0,), start_index_map=(0,))

    def allsum16(v):
        for sh in (8, 4, 2, 1):
            perm = lax.gather(v, (lanes ^ sh)[:, None], dimension_numbers=_dn,
                              slice_sizes=(1,),
                              mode=lax.GatherScatterMode.PROMISE_IN_BOUNDS)
            v = v + perm
        return v

    def fire(c, buf):
        rbuf = rows_v.at[buf]
        fbuf = fcv0 if buf == 0 else fcv1
        sem = sem0 if buf == 0 else sem1
        pltpu.async_copy(emb_hbm.at[idx_v.at[c]], rbuf, sem)
        pltpu.async_copy(fc_hbm.at[idx_v.at[c]], fbuf, sem)

    def drain(c, buf):
        rbuf = rows_v.at[buf]
        fbuf = fcv0 if buf == 0 else fcv1
        sem = sem0 if buf == 0 else sem1
        pltpu.make_async_copy(emb_hbm.at[idx_v.at[c]], rbuf, sem).wait()
        pltpu.make_async_copy(fc_hbm.at[idx_v.at[c]], fbuf, sem).wait()

    def compute(c, buf):
        rbuf = rows_v.at[buf]
        fbuf = fcv0 if buf == 0 else fcv1

        for g in range(GPC):
            def item_body(b, pv, g=g):
                r0 = g * GI * F + b * F
                s = rbuf[r0, :]
                ss = s * s
                for f in range(1, F):
                    e = rbuf[r0 + f, :]
                    s = s + e
                    ss = ss + e * e
                pair = 0.5 * allsum16(s * s - ss)
                return jnp.where(lanes == b, pair, pv)

            pairvec = plsc.parallel_loop(
                0, GI, carry=jnp.zeros((16,), jnp.float32))(item_body)

            rowbase = g * GI * F + lanes * F
            fcs = jnp.zeros((16,), jnp.float32)
            for f in range(F):
                fcs = fcs + plsc.load_gather(fbuf, [rowbase + f])

            z = pairvec + fcs * w + bias
            outbuf[pl.ds(c * CI + g * GI, GI)] = 1.0 / (1.0 + jnp.exp(-z))

    fire(0, 0)

    def pair_body(p, _):
        c0 = 2 * p
        fire(c0 + 1, 1)
        drain(c0, 0)
        compute(c0, 0)

        @pl.when(c0 + 2 < NCH)
        def _():
            fire(c0 + 2, 0)

        drain(c0 + 1, 1)
        compute(c0 + 1, 1)
        return 0

    lax.fori_loop(0, NCH // 2, pair_body, 0)

    pltpu.sync_copy(outbuf, out_hbm.at[pl.ds(wid * ITEMS_W, ITEMS_W)])


def kernel(x, emb_table, fc_table, lin_w, lin_b):
    x3 = x.reshape(NW, NCH, ROWS_C)
    emb_lin = _transpose_sc(emb_table.T)
    wb = jnp.zeros((16,), jnp.float32)
    wb = wb.at[0].set(lin_w[0, 0]).at[1].set(lin_b[0])
    out = _fm_sc(x3, emb_lin.reshape(V, K), fc_table.reshape(-1), wb)
    return out.reshape(B, 1)


# speedup vs baseline: 22.7906x; 1.0048x over previous
import functools

import jax
import jax.numpy as jnp
from jax import lax
from jax.experimental import pallas as pl
from jax.experimental.pallas import tpu as pltpu
from jax.experimental.pallas import tpu_sc as plsc

B = 16384
F = 26
K = 16
NC = 2
NS = 16
NW = NC * NS
ITEMS_W = B // NW
ROWS_W = ITEMS_W * F
GI = 16
CI = 64
ROWS_C = CI * F
NCH = ITEMS_W // CI
GPC = CI // GI

V = 2600000
TCH = 1024
NTCH = V // TCH
TAIL = V - NTCH * TCH
QMAX = (NTCH + 1 + NW - 1) // NW

_mesh = plsc.VectorSubcoreMesh(core_axis_name="c", subcore_axis_name="s")


@functools.partial(
    pl.kernel,
    out_type=jax.ShapeDtypeStruct((V * K,), jnp.float32),
    mesh=_mesh,
    compiler_params=pltpu.CompilerParams(
        needs_layout_passes=False, use_tc_tiling_on_sc=True),
    scratch_types=[
        pltpu.VMEM((8, TCH), jnp.float32),
        pltpu.VMEM((8, TCH), jnp.float32),
        pltpu.VMEM((8, TCH), jnp.float32),
        pltpu.VMEM((8, TCH), jnp.float32),
        pltpu.VMEM((K * TCH,), jnp.float32),
        pltpu.VMEM((K * TCH,), jnp.float32),
        pltpu.VMEM((K, TAIL), jnp.float32),
        pltpu.VMEM((K * TAIL,), jnp.float32),
        pltpu.SemaphoreType.DMA,
        pltpu.SemaphoreType.DMA,
        pltpu.SemaphoreType.DMA,
        pltpu.SemaphoreType.DMA,
    ],
)
def _transpose_sc(embt_hbm, out_hbm, inA0, inB0, inA1, inB1, ob0, ob1,
                  itail, otail, semi0, semi1, semo0, semo1):
    wid = lax.axis_index("s") * NC + lax.axis_index("c")
    lanes = lax.iota(jnp.int32, 16)
    LPC = TCH // 8
    kbase = lanes * TCH

    def bufs(buf):
        if buf == 0:
            return inA0, inB0, ob0, semi0, semo0
        return inA1, inB1, ob1, semi1, semo1

    def fire_in(c, buf):
        inA, inB, obuf, semi, semo = bufs(buf)
        pltpu.async_copy(embt_hbm.at[pl.ds(0, 8), pl.ds(c * TCH, TCH)],
                         inA, semi)
        pltpu.async_copy(embt_hbm.at[pl.ds(8, 8), pl.ds(c * TCH, TCH)],
                         inB, semi)

    def wait_in(c, buf):
        inA, inB, obuf, semi, semo = bufs(buf)
        pltpu.make_async_copy(embt_hbm.at[pl.ds(0, 8), pl.ds(c * TCH, TCH)],
                              inA, semi).wait()
        pltpu.make_async_copy(embt_hbm.at[pl.ds(8, 8), pl.ds(c * TCH, TCH)],
                              inB, semi).wait()

    def wait_out(c, buf):
        inA, inB, obuf, semi, semo = bufs(buf)
        pltpu.make_async_copy(obuf, out_hbm.at[pl.ds(c * K * TCH, K * TCH)],
                              semo).wait()

    def work(c, buf):
        inA, inB, obuf, semi, semo = bufs(buf)

        posbase = lanes * K

        @plsc.parallel_loop(0, TCH // K, unroll=1)
        def _(b):
            for k in range(K):
                ref = inA if k < 8 else inB
                v = ref[k % 8, pl.ds(b * K, K)]
                plsc.store_scatter(obuf, [posbase + (b * 256 + k)], v)

        pltpu.async_copy(obuf, out_hbm.at[pl.ds(c * K * TCH, K * TCH)], semo)

    fire_in(wid, 0)

    def q_step(q, buf):
        c = wid + q * NW
        cn = c + NW

        @pl.when(cn < NTCH)
        def _():
            fire_in(cn, 1 - buf)

        @pl.when(c < NTCH)
        def _():
            wait_in(c, buf)

            @pl.when(c >= 2 * NW)
            def _():
                wait_out(c - 2 * NW, buf)

            work(c, buf)

    def pair_body(p, _):
        q_step(2 * p, 0)
        q_step(2 * p + 1, 1)
        return 0

    lax.fori_loop(0, QMAX // 2, pair_body, 0)

    @pl.when(wid == NTCH % NW)
    def _():
        pltpu.sync_copy(embt_hbm.at[:, pl.ds(NTCH * TCH, TAIL)], itail)

        @plsc.parallel_loop(0, TAIL // 8, unroll=2)
        def _(m):
            for r in range(8):
                iv = jnp.full((16,), m * 8 + r, jnp.int32)
                otail[pl.ds(m * 128 + r * K, K)] = plsc.load_gather(
                    itail, [lanes, iv])

        pltpu.sync_copy(otail, out_hbm.at[pl.ds(NTCH * K * TCH, K * TAIL)])

    qlast = (NTCH - 1 - wid) // NW

    def drain_one(q):
        @pl.when(q >= 0)
        def _():
            c = wid + q * NW

            @pl.when(q % 2 == 0)
            def _():
                wait_out(c, 0)

            @pl.when(q % 2 == 1)
            def _():
                wait_out(c, 1)

    drain_one(qlast)
    drain_one(qlast - 1)


@functools.partial(
    pl.kernel,
    out_type=jax.ShapeDtypeStruct((B,), jnp.float32),
    mesh=_mesh,
    compiler_params=pltpu.CompilerParams(
        needs_layout_passes=False, use_tc_tiling_on_sc=False),
    scratch_types=[
        pltpu.VMEM((NCH, ROWS_C), jnp.int32),
        pltpu.VMEM((2, ROWS_C, K), jnp.float32),
        pltpu.VMEM((ROWS_C,), jnp.float32),
        pltpu.VMEM((ROWS_C,), jnp.float32),
        pltpu.VMEM((ITEMS_W,), jnp.float32),
        pltpu.VMEM((16,), jnp.float32),
        pltpu.SemaphoreType.DMA,
        pltpu.SemaphoreType.DMA,
    ],
)
def _fm_sc(x_hbm, emb_hbm, fc_hbm, wb_hbm, out_hbm,
           idx_v, rows_v, fcv0, fcv1, outbuf, wbv, sem0, sem1):
    wid = lax.axis_index("s") * NC + lax.axis_index("c")

    pltpu.sync_copy(x_hbm.at[wid], idx_v)
    pltpu.sync_copy(wb_hbm, wbv)
    wb16 = wbv[:]
    w = wb16[0]
    bias = wb16[1]

    lanes = lax.iota(jnp.int32, 16)

    _dn = lax.GatherDimensionNumbers(
        offset_dims=(), collapsed_slice_dims=(0,), start_index_map=(0,))

    def allsum16(v):
        for sh in (8, 4, 2, 1):
            perm = lax.gather(v, (lanes ^ sh)[:, None], dimension_numbers=_dn,
                              slice_sizes=(1,),
                              mode=lax.GatherScatterMode.PROMISE_IN_BOUNDS)
            v = v + perm
        return v

    def fire(c, buf):
        rbuf = rows_v.at[buf]
        fbuf = fcv0 if buf == 0 else fcv1
        sem = sem0 if buf == 0 else sem1
        pltpu.async_copy(emb_hbm.at[idx_v.at[c]], rbuf, sem)
        pltpu.async_copy(fc_hbm.at[idx_v.at[c]], fbuf, sem)

    def drain(c, buf):
        rbuf = rows_v.at[buf]
        fbuf = fcv0 if buf == 0 else fcv1
        sem = sem0 if buf == 0 else sem1
        pltpu.make_async_copy(emb_hbm.at[idx_v.at[c]], rbuf, sem).wait()
        pltpu.make_async_copy(fc_hbm.at[idx_v.at[c]], fbuf, sem).wait()

    def compute(c, buf):
        rbuf = rows_v.at[buf]
        fbuf = fcv0 if buf == 0 else fcv1

        for g in range(GPC):
            def item_body(b, pv, g=g):
                r0 = g * GI * F + b * F
                s0 = rbuf[r0, :]
                s1 = rbuf[r0 + 1, :]
                ss0 = s0 * s0
                ss1 = s1 * s1
                for f in range(2, F, 2):
                    e0 = rbuf[r0 + f, :]
                    e1 = rbuf[r0 + f + 1, :]
                    s0 = s0 + e0
                    s1 = s1 + e1
                    ss0 = ss0 + e0 * e0
                    ss1 = ss1 + e1 * e1
                s = s0 + s1
                pair = 0.5 * allsum16(s * s - (ss0 + ss1))
                return jnp.where(lanes == b, pair, pv)

            pairvec = plsc.parallel_loop(
                0, GI, carry=jnp.zeros((16,), jnp.float32))(item_body)

            rowbase = g * GI * F + lanes * F
            fcs = jnp.zeros((16,), jnp.float32)
            for f in range(F):
                fcs = fcs + plsc.load_gather(fbuf, [rowbase + f])

            z = pairvec + fcs * w + bias
            outbuf[pl.ds(c * CI + g * GI, GI)] = 1.0 / (1.0 + jnp.exp(-z))

    fire(0, 0)

    def pair_body(p, _):
        c0 = 2 * p
        fire(c0 + 1, 1)
        drain(c0, 0)
        compute(c0, 0)

        @pl.when(c0 + 2 < NCH)
        def _():
            fire(c0 + 2, 0)

        drain(c0 + 1, 1)
        compute(c0 + 1, 1)
        return 0

    lax.fori_loop(0, NCH // 2, pair_body, 0)

    pltpu.sync_copy(outbuf, out_hbm.at[pl.ds(wid * ITEMS_W, ITEMS_W)])


def kernel(x, emb_table, fc_table, lin_w, lin_b):
    x3 = x.reshape(NW, NCH, ROWS_C)
    emb_lin = _transpose_sc(emb_table.T)
    wb = jnp.zeros((16,), jnp.float32)
    wb = wb.at[0].set(lin_w[0, 0]).at[1].set(lin_b[0])
    out = _fm_sc(x3, emb_lin.reshape(V, K), fc_table.reshape(-1), wb)
    return out.reshape(B, 1)


# speedup vs baseline: 22.9314x; 1.0062x over previous
import functools

import jax
import jax.numpy as jnp
from jax import lax
from jax.experimental import pallas as pl
from jax.experimental.pallas import tpu as pltpu
from jax.experimental.pallas import tpu_sc as plsc

B = 16384
F = 26
K = 16
NC = 2
NS = 16
NW = NC * NS
ITEMS_W = B // NW
ROWS_W = ITEMS_W * F
GI = 16
CI = 64
ROWS_C = CI * F
NCH = ITEMS_W // CI
GPC = CI // GI

V = 2600000
TCH = 2048
NTCH = V // TCH
REM = V - NTCH * TCH
MID = 1024
TAIL = REM - MID
QMAX = (NTCH + NW - 1) // NW
QMAX += QMAX % 2

_mesh = plsc.VectorSubcoreMesh(core_axis_name="c", subcore_axis_name="s")


@functools.partial(
    pl.kernel,
    out_type=jax.ShapeDtypeStruct((V * K,), jnp.float32),
    mesh=_mesh,
    compiler_params=pltpu.CompilerParams(
        needs_layout_passes=False, use_tc_tiling_on_sc=True),
    scratch_types=[
        pltpu.VMEM((8, TCH), jnp.float32),
        pltpu.VMEM((8, TCH), jnp.float32),
        pltpu.VMEM((8, TCH), jnp.float32),
        pltpu.VMEM((8, TCH), jnp.float32),
        pltpu.VMEM((K * TCH,), jnp.float32),
        pltpu.VMEM((K, TAIL), jnp.float32),
        pltpu.VMEM((K * TAIL,), jnp.float32),
        pltpu.SemaphoreType.DMA,
        pltpu.SemaphoreType.DMA,
        pltpu.SemaphoreType.DMA,
    ],
)
def _transpose_sc(embt_hbm, out_hbm, inA0, inB0, inA1, inB1, ob0,
                  itail, otail, semi0, semi1, semo0):
    wid = lax.axis_index("s") * NC + lax.axis_index("c")
    lanes = lax.iota(jnp.int32, 16)
    LPC = TCH // 8
    kbase = lanes * TCH

    def ins(buf):
        if buf == 0:
            return inA0, inB0, semi0
        return inA1, inB1, semi1

    def fire_in(c, buf):
        inA, inB, semi = ins(buf)
        pltpu.async_copy(embt_hbm.at[pl.ds(0, 8), pl.ds(c * TCH, TCH)],
                         inA, semi)
        pltpu.async_copy(embt_hbm.at[pl.ds(8, 8), pl.ds(c * TCH, TCH)],
                         inB, semi)

    def wait_in(c, buf):
        inA, inB, semi = ins(buf)
        pltpu.make_async_copy(embt_hbm.at[pl.ds(0, 8), pl.ds(c * TCH, TCH)],
                              inA, semi).wait()
        pltpu.make_async_copy(embt_hbm.at[pl.ds(8, 8), pl.ds(c * TCH, TCH)],
                              inB, semi).wait()

    def wait_out(c):
        pltpu.make_async_copy(ob0, out_hbm.at[pl.ds(c * K * TCH, K * TCH)],
                              semo0).wait()

    def work(c, buf):
        inA, inB, semi = ins(buf)

        posbase = lanes * K

        @plsc.parallel_loop(0, TCH // K, unroll=1)
        def _(b):
            for k in range(K):
                ref = inA if k < 8 else inB
                v = ref[k % 8, pl.ds(b * K, K)]
                plsc.store_scatter(ob0, [posbase + (b * 256 + k)], v)

        pltpu.async_copy(ob0, out_hbm.at[pl.ds(c * K * TCH, K * TCH)], semo0)

    fire_in(wid, 0)

    def q_step(q, buf):
        c = wid + q * NW
        cn = c + NW

        @pl.when(cn < NTCH)
        def _():
            fire_in(cn, 1 - buf)

        @pl.when(c < NTCH)
        def _():
            wait_in(c, buf)

            @pl.when(c >= NW)
            def _():
                wait_out(c - NW)

            work(c, buf)

    def pair_body(p, _):
        q_step(2 * p, 0)
        q_step(2 * p + 1, 1)
        return 0

    lax.fori_loop(0, QMAX // 2, pair_body, 0)

    qlast = (NTCH - 1 - wid) // NW

    @pl.when(wid < NTCH)
    def _():
        wait_out(wid + qlast * NW)

    @pl.when(wid == (NTCH + 1) % NW)
    def _():
        inA, inB, semi = ins(0)
        mA = inA.at[:, pl.ds(0, MID)]
        mB = inB.at[:, pl.ds(0, MID)]
        pltpu.sync_copy(embt_hbm.at[pl.ds(0, 8), pl.ds(NTCH * TCH, MID)], mA)
        pltpu.sync_copy(embt_hbm.at[pl.ds(8, 8), pl.ds(NTCH * TCH, MID)], mB)
        posbase = lanes * K

        @plsc.parallel_loop(0, MID // K, unroll=1)
        def _(b):
            for k in range(K):
                ref = inA if k < 8 else inB
                v = ref[k % 8, pl.ds(b * K, K)]
                plsc.store_scatter(ob0, [posbase + (b * 256 + k)], v)

        pltpu.sync_copy(ob0.at[pl.ds(0, K * MID)],
                        out_hbm.at[pl.ds(NTCH * K * TCH, K * MID)])

    @pl.when(wid == (NTCH + 2) % NW)
    def _():
        pltpu.sync_copy(embt_hbm.at[:, pl.ds(NTCH * TCH + MID, TAIL)], itail)

        @plsc.parallel_loop(0, TAIL // 8, unroll=2)
        def _(m):
            for r in range(8):
                iv = jnp.full((16,), m * 8 + r, jnp.int32)
                otail[pl.ds(m * 128 + r * K, K)] = plsc.load_gather(
                    itail, [lanes, iv])

        pltpu.sync_copy(
            otail, out_hbm.at[pl.ds(NTCH * K * TCH + K * MID, K * TAIL)])


@functools.partial(
    pl.kernel,
    out_type=jax.ShapeDtypeStruct((B,), jnp.float32),
    mesh=_mesh,
    compiler_params=pltpu.CompilerParams(
        needs_layout_passes=False, use_tc_tiling_on_sc=False),
    scratch_types=[
        pltpu.VMEM((NCH, ROWS_C), jnp.int32),
        pltpu.VMEM((2, ROWS_C, K), jnp.float32),
        pltpu.VMEM((ROWS_C,), jnp.float32),
        pltpu.VMEM((ROWS_C,), jnp.float32),
        pltpu.VMEM((ITEMS_W,), jnp.float32),
        pltpu.VMEM((16,), jnp.float32),
        pltpu.SemaphoreType.DMA,
        pltpu.SemaphoreType.DMA,
    ],
)
def _fm_sc(x_hbm, emb_hbm, fc_hbm, wb_hbm, out_hbm,
           idx_v, rows_v, fcv0, fcv1, outbuf, wbv, sem0, sem1):
    wid = lax.axis_index("s") * NC + lax.axis_index("c")

    pltpu.sync_copy(x_hbm.at[wid], idx_v)
    pltpu.sync_copy(wb_hbm, wbv)
    wb16 = wbv[:]
    w = wb16[0]
    bias = wb16[1]

    lanes = lax.iota(jnp.int32, 16)

    _dn = lax.GatherDimensionNumbers(
        offset_dims=(), collapsed_slice_dims=(0,), start_index_map=(0,))

    def allsum16(v):
        for sh in (8, 4, 2, 1):
            perm = lax.gather(v, (lanes ^ sh)[:, None], dimension_numbers=_dn,
                              slice_sizes=(1,),
                              mode=lax.GatherScatterMode.PROMISE_IN_BOUNDS)
            v = v + perm
        return v

    def fire(c, buf):
        rbuf = rows_v.at[buf]
        fbuf = fcv0 if buf == 0 else fcv1
        sem = sem0 if buf == 0 else sem1
        pltpu.async_copy(emb_hbm.at[idx_v.at[c]], rbuf, sem)
        pltpu.async_copy(fc_hbm.at[idx_v.at[c]], fbuf, sem)

    def drain(c, buf):
        rbuf = rows_v.at[buf]
        fbuf = fcv0 if buf == 0 else fcv1
        sem = sem0 if buf == 0 else sem1
        pltpu.make_async_copy(emb_hbm.at[idx_v.at[c]], rbuf, sem).wait()
        pltpu.make_async_copy(fc_hbm.at[idx_v.at[c]], fbuf, sem).wait()

    def compute(c, buf):
        rbuf = rows_v.at[buf]
        fbuf = fcv0 if buf == 0 else fcv1

        for g in range(GPC):
            def item_body(b, pv, g=g):
                r0 = g * GI * F + b * F
                s0 = rbuf[r0, :]
                s1 = rbuf[r0 + 1, :]
                ss0 = s0 * s0
                ss1 = s1 * s1
                for f in range(2, F, 2):
                    e0 = rbuf[r0 + f, :]
                    e1 = rbuf[r0 + f + 1, :]
                    s0 = s0 + e0
                    s1 = s1 + e1
                    ss0 = ss0 + e0 * e0
                    ss1 = ss1 + e1 * e1
                s = s0 + s1
                pair = 0.5 * allsum16(s * s - (ss0 + ss1))
                return jnp.where(lanes == b, pair, pv)

            pairvec = plsc.parallel_loop(
                0, GI, carry=jnp.zeros((16,), jnp.float32))(item_body)

            rowbase = g * GI * F + lanes * F
            fcs = jnp.zeros((16,), jnp.float32)
            for f in range(F):
                fcs = fcs + plsc.load_gather(fbuf, [rowbase + f])

            z = pairvec + fcs * w + bias
            outbuf[pl.ds(c * CI + g * GI, GI)] = 1.0 / (1.0 + jnp.exp(-z))

    fire(0, 0)

    def pair_body(p, _):
        c0 = 2 * p
        fire(c0 + 1, 1)
        drain(c0, 0)
        compute(c0, 0)

        @pl.when(c0 + 2 < NCH)
        def _():
            fire(c0 + 2, 0)

        drain(c0 + 1, 1)
        compute(c0 + 1, 1)
        return 0

    lax.fori_loop(0, NCH // 2, pair_body, 0)

    pltpu.sync_copy(outbuf, out_hbm.at[pl.ds(wid * ITEMS_W, ITEMS_W)])


def kernel(x, emb_table, fc_table, lin_w, lin_b):
    x3 = x.reshape(NW, NCH, ROWS_C)
    emb_lin = _transpose_sc(emb_table.T)
    wb = jnp.zeros((16,), jnp.float32)
    wb = wb.at[0].set(lin_w[0, 0]).at[1].set(lin_b[0])
    out = _fm_sc(x3, emb_lin.reshape(V, K), fc_table.reshape(-1), wb)
    return out.reshape(B, 1)
